# baseline (speedup 1.0000x reference)
import jax
import jax.numpy as jnp
from jax.experimental import pallas as pl
from jax.experimental.pallas import tpu as pltpu

_NEG_INF = -1e30
_BB = 128
_BS = 128


def _conv1_pool1_kernel(x_ref, w1_ref, b1_ref, o_ref):
    x = x_ref[...]
    for co in range(20):
        zs = []
        for r in (0, 1):
            for s in (0, 1):
                acc = None
                for k in range(25):
                    di, dj = k // 5, k % 5
                    pi, oi = (r + di) % 2, (r + di) // 2
                    pj, oj = (s + dj) % 2, (s + dj) // 2
                    t = w1_ref[co, k] * x[pi, pj, oi:oi + 12, oj:oj + 12, :]
                    acc = t if acc is None else acc + t
                zs.append(acc)
        p = jnp.maximum(jnp.maximum(zs[0], zs[1]),
                        jnp.maximum(zs[2], zs[3]))
        o_ref[co] = jnp.maximum(p + b1_ref[co], 0.0).astype(jnp.bfloat16)


def _conv2_head_kernel(p_ref, w2_ref, b2_ref, wf1_ref, bf1_ref,
                       wf2_ref, bf2_ref, o_ref, acc_ref):
    for t in range(25):
        di, dj = t // 5, t % 5
        xp = p_ref[:, di:di + 8, dj:dj + 8, :].reshape(_BS * 64, 20)
        z = jnp.dot(xp, w2_ref[t], preferred_element_type=jnp.float32)
        if t == 0:
            acc_ref[...] = z
        else:
            acc_ref[...] = acc_ref[...] + z

    a3 = acc_ref[...].reshape(_BS, 8, 8, 128)
    pieces = []
    for i in range(4):
        rm = jnp.maximum(a3[:, 2 * i], a3[:, 2 * i + 1])
        for j in range(4):
            pieces.append(jnp.maximum(rm[:, 2 * j:2 * j + 1, :],
                                      rm[:, 2 * j + 1:2 * j + 2, :]))
    p2 = jnp.concatenate(pieces, axis=1)
    pooled = jnp.maximum(p2 + b2_ref[0], 0.0).astype(jnp.bfloat16)

    h = jnp.dot(pooled[:, 0, :], wf1_ref[0],
                preferred_element_type=jnp.float32)
    for s in range(1, 16):
        h = h + jnp.dot(pooled[:, s, :], wf1_ref[s],
                        preferred_element_type=jnp.float32)
    h = jnp.maximum(h + bf1_ref[0], 0.0).astype(jnp.bfloat16)

    logits = jnp.dot(h, wf2_ref[...],
                     preferred_element_type=jnp.float32) + bf2_ref[0]
    mx = jnp.max(logits, axis=-1, keepdims=True)
    lse = mx + jnp.log(jnp.sum(jnp.exp(logits - mx), axis=-1, keepdims=True))
    o_ref[...] = logits - lse


def kernel(x, w1, b1, w2, b2, w_fc1, b_fc1, w_fc2, b_fc2):
    N = x.shape[0]
    classes = w_fc2.shape[1]

    xT = jnp.transpose(x.reshape(N, 14, 2, 14, 2), (2, 4, 1, 3, 0))
    w1m = w1.reshape(20, 25)

    cparams = pltpu.CompilerParams(dimension_semantics=("parallel",),
                                   vmem_limit_bytes=64 * 1024 * 1024)

    p1 = pl.pallas_call(
        _conv1_pool1_kernel,
        out_shape=jax.ShapeDtypeStruct((20, 12, 12, N), jnp.bfloat16),
        grid=(N // _BB,),
        in_specs=[
            pl.BlockSpec((2, 2, 14, 14, _BB), lambda b: (0, 0, 0, 0, b)),
            pl.BlockSpec(memory_space=pltpu.SMEM),
            pl.BlockSpec(memory_space=pltpu.SMEM),
        ],
        out_specs=pl.BlockSpec((20, 12, 12, _BB), lambda b: (0, 0, 0, b)),
        compiler_params=cparams,
    )(xT, w1m, b1)

    p1t = jnp.transpose(p1, (3, 1, 2, 0))

    w2p = jnp.pad(jnp.transpose(w2, (2, 3, 1, 0)).reshape(25, 20, 50),
                  ((0, 0), (0, 0), (0, 128 - 50))).astype(jnp.bfloat16)
    b2p = jnp.pad(b2, (0, 128 - 50)).reshape(1, 128)
    wf1 = jnp.transpose(w_fc1.reshape(50, 16, 500), (1, 0, 2))
    wf1 = jnp.pad(wf1, ((0, 0), (0, 128 - 50), (0, 12))).astype(jnp.bfloat16)
    bf1 = jnp.pad(b_fc1, (0, 12)).reshape(1, 512)
    wf2 = jnp.pad(w_fc2, ((0, 12), (0, 128 - classes))).astype(jnp.bfloat16)
    bf2 = jnp.pad(b_fc2, (0, 128 - classes),
                  constant_values=_NEG_INF).reshape(1, 128)

    out = pl.pallas_call(
        _conv2_head_kernel,
        out_shape=jax.ShapeDtypeStruct((N, 128), jnp.float32),
        grid=(N // _BS,),
        in_specs=[
            pl.BlockSpec((_BS, 12, 12, 20), lambda m: (m, 0, 0, 0)),
            pl.BlockSpec((25, 20, 128), lambda m: (0, 0, 0)),
            pl.BlockSpec((1, 128), lambda m: (0, 0)),
            pl.BlockSpec((16, 128, 512), lambda m: (0, 0, 0)),
            pl.BlockSpec((1, 512), lambda m: (0, 0)),
            pl.BlockSpec((512, 128), lambda m: (0, 0)),
            pl.BlockSpec((1, 128), lambda m: (0, 0)),
        ],
        out_specs=pl.BlockSpec((_BS, 128), lambda m: (m, 0)),
        scratch_shapes=[pltpu.VMEM((_BS * 64, 128), jnp.float32)],
        compiler_params=cparams,
    )(p1t, w2p, b2p, wf1, bf1, wf2, bf2)

    return out[:, :classes]


# speedup vs baseline: 1.0030x; 1.0030x over previous
import jax
import jax.numpy as jnp
from jax.experimental import pallas as pl
from jax.experimental.pallas import tpu as pltpu

_NEG_INF = -1e30
_BB = 128
_BS = 64


def _conv1_pool1_kernel(x_ref, w1_ref, b1_ref, o_ref):
    x = x_ref[...]
    for co in range(20):
        zs = []
        for r in (0, 1):
            for s in (0, 1):
                acc = None
                for k in range(25):
                    di, dj = k // 5, k % 5
                    pi, oi = (r + di) % 2, (r + di) // 2
                    pj, oj = (s + dj) % 2, (s + dj) // 2
                    t = w1_ref[co, k] * x[pi, pj, oi:oi + 12, oj:oj + 12, :]
                    acc = t if acc is None else acc + t
                zs.append(acc)
        p = jnp.maximum(jnp.maximum(zs[0], zs[1]),
                        jnp.maximum(zs[2], zs[3]))
        o_ref[co] = jnp.maximum(p + b1_ref[co], 0.0).astype(jnp.bfloat16)


def _conv2_head_kernel(p_ref, w2_ref, b2_ref, wf1_ref, bf1_ref,
                       wf2_ref, bf2_ref, o_ref, acc_ref):
    for t in range(25):
        di, dj = t // 5, t % 5
        xp = p_ref[:, di:di + 8, dj:dj + 8, :].reshape(_BS * 64, 20)
        z = jnp.dot(xp, w2_ref[t], preferred_element_type=jnp.float32)
        if t == 0:
            acc_ref[...] = z
        else:
            acc_ref[...] = acc_ref[...] + z

    a3 = acc_ref[...].reshape(_BS, 8, 8, 128)
    pieces = []
    for i in range(4):
        rm = jnp.maximum(a3[:, 2 * i], a3[:, 2 * i + 1])
        for j in range(4):
            pieces.append(jnp.maximum(rm[:, 2 * j:2 * j + 1, :],
                                      rm[:, 2 * j + 1:2 * j + 2, :]))
    p2 = jnp.concatenate(pieces, axis=1)
    pooled = jnp.maximum(p2 + b2_ref[0], 0.0).astype(jnp.bfloat16)

    h = jnp.dot(pooled[:, 0, :], wf1_ref[0],
                preferred_element_type=jnp.float32)
    for s in range(1, 16):
        h = h + jnp.dot(pooled[:, s, :], wf1_ref[s],
                        preferred_element_type=jnp.float32)
    h = jnp.maximum(h + bf1_ref[0], 0.0).astype(jnp.bfloat16)

    logits = jnp.dot(h, wf2_ref[...],
                     preferred_element_type=jnp.float32) + bf2_ref[0]
    mx = jnp.max(logits, axis=-1, keepdims=True)
    lse = mx + jnp.log(jnp.sum(jnp.exp(logits - mx), axis=-1, keepdims=True))
    o_ref[...] = logits - lse


def kernel(x, w1, b1, w2, b2, w_fc1, b_fc1, w_fc2, b_fc2):
    N = x.shape[0]
    classes = w_fc2.shape[1]

    xT = jnp.transpose(x.reshape(N, 14, 2, 14, 2), (2, 4, 1, 3, 0))
    w1m = w1.reshape(20, 25)

    cparams = pltpu.CompilerParams(dimension_semantics=("parallel",),
                                   vmem_limit_bytes=64 * 1024 * 1024)

    p1 = pl.pallas_call(
        _conv1_pool1_kernel,
        out_shape=jax.ShapeDtypeStruct((20, 12, 12, N), jnp.bfloat16),
        grid=(N // _BB,),
        in_specs=[
            pl.BlockSpec((2, 2, 14, 14, _BB), lambda b: (0, 0, 0, 0, b)),
            pl.BlockSpec(memory_space=pltpu.SMEM),
            pl.BlockSpec(memory_space=pltpu.SMEM),
        ],
        out_specs=pl.BlockSpec((20, 12, 12, _BB), lambda b: (0, 0, 0, b)),
        compiler_params=cparams,
    )(xT, w1m, b1)

    p1t = jnp.transpose(p1, (3, 1, 2, 0))

    w2p = jnp.pad(jnp.transpose(w2, (2, 3, 1, 0)).reshape(25, 20, 50),
                  ((0, 0), (0, 0), (0, 128 - 50))).astype(jnp.bfloat16)
    b2p = jnp.pad(b2, (0, 128 - 50)).reshape(1, 128)
    wf1 = jnp.transpose(w_fc1.reshape(50, 16, 500), (1, 0, 2))
    wf1 = jnp.pad(wf1, ((0, 0), (0, 128 - 50), (0, 12))).astype(jnp.bfloat16)
    bf1 = jnp.pad(b_fc1, (0, 12)).reshape(1, 512)
    wf2 = jnp.pad(w_fc2, ((0, 12), (0, 128 - classes))).astype(jnp.bfloat16)
    bf2 = jnp.pad(b_fc2, (0, 128 - classes),
                  constant_values=_NEG_INF).reshape(1, 128)

    out = pl.pallas_call(
        _conv2_head_kernel,
        out_shape=jax.ShapeDtypeStruct((N, 128), jnp.float32),
        grid=(N // _BS,),
        in_specs=[
            pl.BlockSpec((_BS, 12, 12, 20), lambda m: (m, 0, 0, 0)),
            pl.BlockSpec((25, 20, 128), lambda m: (0, 0, 0)),
            pl.BlockSpec((1, 128), lambda m: (0, 0)),
            pl.BlockSpec((16, 128, 512), lambda m: (0, 0, 0)),
            pl.BlockSpec((1, 512), lambda m: (0, 0)),
            pl.BlockSpec((512, 128), lambda m: (0, 0)),
            pl.BlockSpec((1, 128), lambda m: (0, 0)),
        ],
        out_specs=pl.BlockSpec((_BS, 128), lambda m: (m, 0)),
        scratch_shapes=[pltpu.VMEM((_BS * 64, 128), jnp.float32)],
        compiler_params=cparams,
    )(p1t, w2p, b2p, wf1, bf1, wf2, bf2)

    return out[:, :classes]


# speedup vs baseline: 1.7258x; 1.7207x over previous
import jax
import jax.numpy as jnp
from jax.experimental import pallas as pl
from jax.experimental.pallas import tpu as pltpu

_NEG_INF = -1e30
_BB = 128
_BS = 64


def _conv1_pool1_kernel(x_ref, w1_ref, b1_ref, o_ref):
    x = x_ref[...]
    for co in range(20):
        part = {}
        for r in (0, 1):
            for dj in range(5):
                acc = None
                for di in range(5):
                    pi, oi = (r + di) % 2, (r + di) // 2
                    pj = dj % 2
                    t = w1_ref[co, di * 5 + dj] * x[pi, pj, oi:oi + 12, :, :]
                    acc = t if acc is None else acc + t
                part[(r, dj)] = acc
        zs = []
        for r in (0, 1):
            for s in (0, 1):
                acc = None
                for dj in range(5):
                    oj = (s + dj) // 2
                    t = part[(r, dj)][:, oj:oj + 12, :]
                    acc = t if acc is None else acc + t
                zs.append(acc)
        p = jnp.maximum(jnp.maximum(zs[0], zs[1]),
                        jnp.maximum(zs[2], zs[3]))
        o_ref[co] = jnp.maximum(p + b1_ref[co], 0.0)


def _conv2_head_kernel(p_ref, w2_ref, b2_ref, wf1_ref, bf1_ref,
                       wf2_ref, bf2_ref, o_ref, acc_ref):
    for t in range(25):
        di, dj = t // 5, t % 5
        xp = p_ref[:, di:di + 8, dj:dj + 8, :].reshape(_BS * 64, 20)
        z = jnp.dot(xp, w2_ref[t], preferred_element_type=jnp.float32)
        if t == 0:
            acc_ref[...] = z
        else:
            acc_ref[...] = acc_ref[...] + z

    a3 = acc_ref[...].reshape(_BS, 8, 8, 128)
    pieces = []
    for i in range(4):
        rm = jnp.maximum(a3[:, 2 * i], a3[:, 2 * i + 1])
        for j in range(4):
            pieces.append(jnp.maximum(rm[:, 2 * j:2 * j + 1, :],
                                      rm[:, 2 * j + 1:2 * j + 2, :]))
    p2 = jnp.concatenate(pieces, axis=1)
    pooled = jnp.maximum(p2 + b2_ref[0], 0.0)

    h = jnp.dot(pooled[:, 0, :], wf1_ref[0],
                preferred_element_type=jnp.float32)
    for s in range(1, 16):
        h = h + jnp.dot(pooled[:, s, :], wf1_ref[s],
                        preferred_element_type=jnp.float32)
    h = jnp.maximum(h + bf1_ref[0], 0.0)

    logits = jnp.dot(h, wf2_ref[...],
                     preferred_element_type=jnp.float32) + bf2_ref[0]
    mx = jnp.max(logits, axis=-1, keepdims=True)
    lse = mx + jnp.log(jnp.sum(jnp.exp(logits - mx), axis=-1, keepdims=True))
    o_ref[...] = logits - lse


def kernel(x, w1, b1, w2, b2, w_fc1, b_fc1, w_fc2, b_fc2):
    N = x.shape[0]
    classes = w_fc2.shape[1]

    xT = jnp.transpose(x.reshape(N, 14, 2, 14, 2), (2, 4, 1, 3, 0))
    w1m = w1.reshape(20, 25)

    cparams = pltpu.CompilerParams(dimension_semantics=("parallel",),
                                   vmem_limit_bytes=64 * 1024 * 1024)

    p1 = pl.pallas_call(
        _conv1_pool1_kernel,
        out_shape=jax.ShapeDtypeStruct((20, 12, 12, N), jnp.float32),
        grid=(N // _BB,),
        in_specs=[
            pl.BlockSpec((2, 2, 14, 14, _BB), lambda b: (0, 0, 0, 0, b)),
            pl.BlockSpec(memory_space=pltpu.SMEM),
            pl.BlockSpec(memory_space=pltpu.SMEM),
        ],
        out_specs=pl.BlockSpec((20, 12, 12, _BB), lambda b: (0, 0, 0, b)),
        compiler_params=cparams,
    )(xT, w1m, b1)

    p1t = jnp.transpose(p1, (3, 1, 2, 0))

    w2p = jnp.pad(jnp.transpose(w2, (2, 3, 1, 0)).reshape(25, 20, 50),
                  ((0, 0), (0, 0), (0, 128 - 50)))
    b2p = jnp.pad(b2, (0, 128 - 50)).reshape(1, 128)
    wf1 = jnp.transpose(w_fc1.reshape(50, 16, 500), (1, 0, 2))
    wf1 = jnp.pad(wf1, ((0, 0), (0, 128 - 50), (0, 12)))
    bf1 = jnp.pad(b_fc1, (0, 12)).reshape(1, 512)
    wf2 = jnp.pad(w_fc2, ((0, 12), (0, 128 - classes)))
    bf2 = jnp.pad(b_fc2, (0, 128 - classes),
                  constant_values=_NEG_INF).reshape(1, 128)

    out = pl.pallas_call(
        _conv2_head_kernel,
        out_shape=jax.ShapeDtypeStruct((N, 128), jnp.float32),
        grid=(N // _BS,),
        in_specs=[
            pl.BlockSpec((_BS, 12, 12, 20), lambda m: (m, 0, 0, 0)),
            pl.BlockSpec((25, 20, 128), lambda m: (0, 0, 0)),
            pl.BlockSpec((1, 128), lambda m: (0, 0)),
            pl.BlockSpec((16, 128, 512), lambda m: (0, 0, 0)),
            pl.BlockSpec((1, 512), lambda m: (0, 0)),
            pl.BlockSpec((512, 128), lambda m: (0, 0)),
            pl.BlockSpec((1, 128), lambda m: (0, 0)),
        ],
        out_specs=pl.BlockSpec((_BS, 128), lambda m: (m, 0)),
        scratch_shapes=[pltpu.VMEM((_BS * 64, 128), jnp.float32)],
        compiler_params=cparams,
    )(p1t, w2p, b2p, wf1, bf1, wf2, bf2)

    return out[:, :classes]
